# Initial kernel scaffold; baseline (speedup 1.0000x reference)
import jax
import jax.numpy as jnp
from jax.experimental import pallas as pl


def kernel(tokens, attention_x, attention_y):
    raise NotImplementedError("write your pallas kernel here")


# speedup vs baseline: 1.3541x; 1.3541x over previous
import math

import jax
import jax.numpy as jnp
from jax import lax
from jax.experimental import pallas as pl
from jax.experimental.pallas import tpu as pltpu
from jax.experimental.pallas import tpu_sc as plsc

B, N, C = 16, 8192, 512
K = math.ceil(N * 0.6)
NK = N - K

NW = 32
CH = 64
BLKS = B * K // 8
BLK_BIG = BLKS // NW + 1
BLK_SMALL = BLKS // NW
BIGW = BLKS - BLK_SMALL * NW
ROWS_BIG = BLK_BIG * 8
ROWS_SMALL = BLK_SMALL * 8
KC_FULL = 38
KT_BIG = ROWS_BIG - KC_FULL * CH
KT_SMALL = ROWS_SMALL - KC_FULL * CH
KC = KC_FULL + 1
NH = NK // 2
NC_ = (NH + CH - 1) // CH
NHP = NC_ * CH
CV = C // 16


def _sc_body(tokens_hbm, kidx_hbm, nidx_hbm, w_hbm, ax_hbm, ay_hbm,
             cutf_hbm, icut_hbm,
             sel_out, extra_out, mask_out,
             kidx_v, nidx_v, w_v, buf, acc_v, maskbuf, sv_a, sv_b,
             cutf_v, icut_v, shacc, in_sem, out_sem):
    c = lax.axis_index("c")
    s = lax.axis_index("s")
    w = c * 16 + s
    b = c * 8 + s // 2
    r = s % 2
    zero16 = jnp.zeros((16,), jnp.int32)
    iota16 = lax.iota(jnp.int32, 16)

    pltpu.sync_copy(kidx_hbm.at[w], kidx_v)
    pltpu.sync_copy(nidx_hbm.at[b, r], nidx_v)
    pltpu.sync_copy(w_hbm.at[b, r], w_v)
    pltpu.sync_copy(cutf_hbm.at[b], cutf_v)
    pltpu.sync_copy(icut_hbm.at[b], icut_v)

    pltpu.sync_copy(ax_hbm.at[b, r], sv_a)
    pltpu.sync_copy(ay_hbm.at[b, r], sv_b)
    t = cutf_v[0]
    ic = icut_v[0]
    col0 = r * 4096

    def mrow(v, carry):
        off = pl.multiple_of(v * 16, 16)
        sc = sv_a[0, pl.ds(off, 16)] + sv_b[0, pl.ds(off, 16)]
        idx = iota16 + (col0 + v * 16)
        keep = (sc > t) | ((sc == t) & (idx >= ic))
        maskbuf[0, pl.ds(off, 16)] = jnp.where(keep, jnp.float32(1.0),
                                               jnp.float32(0.0))
        return carry
    lax.fori_loop(0, 4096 // 16, mrow, 0)
    pltpu.sync_copy(maskbuf, mask_out.at[b, r])

    def wait_in():
        pltpu.make_async_copy(tokens_hbm.at[pl.ds(0, CH)], buf.at[0],
                              in_sem).wait()

    def wait_out():
        pltpu.make_async_copy(buf.at[0], sel_out.at[pl.ds(0, CH)],
                              out_sem).wait()

    start = pl.multiple_of(
        8 * jnp.where(w < BIGW, BLK_BIG * w,
                      BLK_BIG * BIGW + BLK_SMALL * (w - BIGW)), 8)
    pltpu.async_copy(tokens_hbm.at[kidx_v.at[0]], buf.at[0], in_sem)

    def kbody(j, carry):
        jj = j & 1
        wait_in()

        @pl.when(j >= 1)
        def _():
            wait_out()

        @pl.when(j + 1 < KC)
        def _():
            pltpu.async_copy(tokens_hbm.at[kidx_v.at[j + 1]],
                             buf.at[1 - jj], in_sem)
        off = pl.multiple_of(start + j * CH, 8)
        pltpu.async_copy(buf.at[jj], sel_out.at[pl.ds(off, CH)], out_sem)
        return carry
    lax.fori_loop(0, KC_FULL, kbody, 0)
    wait_in()
    wait_out()
    tail = KC_FULL & 1
    toff = pl.multiple_of(start + KC_FULL * CH, 8)

    @pl.when(w < BIGW)
    def _():
        pltpu.sync_copy(buf.at[tail, pl.ds(0, KT_BIG)],
                        sel_out.at[pl.ds(toff, KT_BIG)])

    @pl.when(w >= BIGW)
    def _():
        pltpu.sync_copy(buf.at[tail, pl.ds(0, KT_SMALL)],
                        sel_out.at[pl.ds(toff, KT_SMALL)])

    for v in range(CV):
        acc_v[0, pl.ds(v * 16, 16)] = jnp.zeros((16,), jnp.float32)
    pltpu.async_copy(tokens_hbm.at[nidx_v.at[0]], buf.at[0], in_sem)

    def nbody(j, carry):
        jj = j & 1
        wait_in()

        @pl.when(j + 1 < NC_)
        def _():
            pltpu.async_copy(tokens_hbm.at[nidx_v.at[j + 1]],
                             buf.at[1 - jj], in_sem)

        def rbody(rr, carry2):
            wv = plsc.load_gather(w_v, [zero16 + j, zero16 + rr])
            for v in range(CV):
                row = buf[jj, rr, pl.ds(v * 16, 16)]
                plsc.addupdate(acc_v.at[0, pl.ds(v * 16, 16)], row * wv)
            return carry2
        lax.fori_loop(0, CH, rbody, 0)
        return carry
    lax.fori_loop(0, NC_, nbody, 0)

    pltpu.sync_copy(acc_v, shacc.at[s])
    plsc.subcore_barrier()

    @pl.when(r == 0)
    def _():
        pltpu.sync_copy(shacc.at[s + 1], maskbuf.at[pl.ds(0, 1), pl.ds(0, C)])
        for v in range(CV):
            acc_v[0, pl.ds(v * 16, 16)] = (acc_v[0, pl.ds(v * 16, 16)]
                                           + maskbuf[0, pl.ds(v * 16, 16)])
        pltpu.sync_copy(acc_v, extra_out.at[b])


_mesh = plsc.VectorSubcoreMesh(core_axis_name="c", subcore_axis_name="s")

_sc_call = pl.kernel(
    _sc_body, mesh=_mesh,
    out_type=[jax.ShapeDtypeStruct((B * K, C), jnp.float32),
              jax.ShapeDtypeStruct((B, 1, C), jnp.float32),
              jax.ShapeDtypeStruct((B, 2, 1, 4096), jnp.float32)],
    scratch_types=[pltpu.VMEM((KC, CH), jnp.int32),
                   pltpu.VMEM((NC_, CH), jnp.int32),
                   pltpu.VMEM((NC_, CH), jnp.float32),
                   pltpu.VMEM((2, CH, C), jnp.float32),
                   pltpu.VMEM((1, C), jnp.float32),
                   pltpu.VMEM((1, 4096), jnp.float32),
                   pltpu.VMEM((1, 4096), jnp.float32),
                   pltpu.VMEM((1, 4096), jnp.float32),
                   pltpu.VMEM((1, 16), jnp.float32),
                   pltpu.VMEM((1, 16), jnp.int32),
                   pltpu.VMEM_SHARED((16, 1, C), jnp.float32),
                   pltpu.SemaphoreType.DMA,
                   pltpu.SemaphoreType.DMA],
    compiler_params=pltpu.CompilerParams(needs_layout_passes=False),
)


def kernel(tokens, attention_x, attention_y):
    score = attention_x + attention_y
    order = jnp.argsort(score, axis=1)[:, ::-1]
    score_sort = jnp.take_along_axis(score, order, axis=1)
    w = jax.nn.softmax(score_sort[:, K:], axis=1)

    base = (jnp.arange(B, dtype=jnp.int32) * N)[:, None]

    gflat = (order[:, :K].astype(jnp.int32) + base).reshape(B * K)
    pieces = []
    start = 0
    for wid in range(NW):
        rows = ROWS_BIG if wid < BIGW else ROWS_SMALL
        pad = KC * CH - rows
        pieces.append(jnp.concatenate(
            [gflat[start:start + rows],
             jnp.arange(pad, dtype=jnp.int32) + (wid * 97 % N)], axis=0))
        start += rows
    kidx = jnp.stack(pieces).reshape(NW, KC, CH)

    norder = order[:, K:].astype(jnp.int32) + base
    npad = jnp.broadcast_to(base + jnp.arange(NHP - NH, dtype=jnp.int32)[None],
                            (B, NHP - NH))
    n0 = jnp.concatenate([norder[:, :NH], npad], axis=1)
    n1 = jnp.concatenate([norder[:, NH:], npad], axis=1)
    nidx = jnp.stack([n0, n1], axis=1).reshape(B, 2, NC_, CH)

    wz = jnp.zeros((B, NHP - NH), jnp.float32)
    wsp = jnp.stack(
        [jnp.concatenate([w[:, :NH], wz], axis=1),
         jnp.concatenate([w[:, NH:], wz], axis=1)],
        axis=1).reshape(B, 2, NC_, CH)

    cutf = jnp.broadcast_to(score_sort[:, K - 1:K], (B, 16)).reshape(B, 1, 16)
    icut = jnp.broadcast_to(order[:, K - 1:K].astype(jnp.int32),
                            (B, 16)).reshape(B, 1, 16)

    sel, extra, mask = _sc_call(
        tokens.reshape(B * N, C), kidx, nidx, wsp,
        attention_x.reshape(B, 2, 1, 4096), attention_y.reshape(B, 2, 1, 4096),
        cutf, icut)
    return (sel.reshape(B, K, C), extra.reshape(B, 1, C), mask.reshape(B, N))
